# Initial kernel scaffold; baseline (speedup 1.0000x reference)
import jax
import jax.numpy as jnp
from jax.experimental import pallas as pl


def kernel(inputs, token_table, pos_table):
    raise NotImplementedError("write your pallas kernel here")


# speedup vs baseline: 1.4174x; 1.4174x over previous
import functools

import jax
import jax.numpy as jnp
from jax import lax
from jax.experimental import pallas as pl
from jax.experimental.pallas import tpu as pltpu
from jax.experimental.pallas import tpu_sc as plsc

MAXLEN = 200
EMBED = 32
BATCH = 4096
ROWS = BATCH * MAXLEN

NC = 2
NSUB = 16
NW = NC * NSUB
RW = ROWS // NW

S = 800
G = 80
NG = S // G
NCH = RW // S
LANES = 16


def _make_sc_kernel():
    mesh = plsc.VectorSubcoreMesh(core_axis_name="c", subcore_axis_name="s")

    @functools.partial(
        pl.kernel,
        mesh=mesh,
        compiler_params=pltpu.CompilerParams(use_tc_tiling_on_sc=False),
        out_type=jax.ShapeDtypeStruct((ROWS, EMBED), jnp.float32),
        scratch_types=[
            pltpu.VMEM((RW // G, G), jnp.int32),
            pltpu.VMEM((MAXLEN, EMBED), jnp.float32),
            pltpu.VMEM((S, EMBED), jnp.float32),
            pltpu.SemaphoreType.DMA,
        ],
    )
    def emb_kernel(idx_hbm, tok_hbm, pos_hbm, out_hbm, idx_v, pos_v, rows_v, sem):
        wid = lax.axis_index("s") * NC + lax.axis_index("c")
        idx_row0 = wid * (RW // G)
        row0 = wid * RW

        pltpu.sync_copy(idx_hbm.at[pl.ds(idx_row0, RW // G)], idx_v)
        pltpu.sync_copy(pos_hbm, pos_v)

        def chunk_body(g, _):
            copies = []
            for k in range(NG):
                copies.append(
                    pltpu.async_copy(
                        tok_hbm.at[idx_v.at[g * NG + k]],
                        rows_v.at[pl.ds(k * G, G)],
                        sem,
                    )
                )
            for c in copies:
                c.wait()

            def add_body(p, _):
                pv0 = pos_v[p, pl.ds(0, LANES)]
                pv1 = pos_v[p, pl.ds(LANES, LANES)]
                for k in range(S // MAXLEN):
                    r = p + k * MAXLEN
                    rows_v[r, pl.ds(0, LANES)] = rows_v[r, pl.ds(0, LANES)] + pv0
                    rows_v[r, pl.ds(LANES, LANES)] = (
                        rows_v[r, pl.ds(LANES, LANES)] + pv1
                    )
                return _

            lax.fori_loop(0, MAXLEN, add_body, None)

            pltpu.sync_copy(rows_v, out_hbm.at[pl.ds(row0 + g * S, S)])
            return _

        lax.fori_loop(0, NCH, chunk_body, None)

    return emb_kernel


_EMB_KERNEL = _make_sc_kernel()


def kernel(inputs, token_table, pos_table):
    idx = inputs.astype(jnp.int32).reshape(ROWS // G, G)
    out = _EMB_KERNEL(idx, token_table, pos_table)
    return out.reshape(BATCH, MAXLEN, EMBED)


# speedup vs baseline: 1.4907x; 1.0517x over previous
import functools

import jax
import jax.numpy as jnp
from jax import lax
from jax.experimental import pallas as pl
from jax.experimental.pallas import tpu as pltpu
from jax.experimental.pallas import tpu_sc as plsc

MAXLEN = 200
EMBED = 32
BATCH = 4096
ROWS = BATCH * MAXLEN

NC = 2
NSUB = 16
NW = NC * NSUB
RW = ROWS // NW

S = 800
G = 80
NG = S // G
NCH = RW // S
LANES = 16


def _make_sc_kernel():
    mesh = plsc.VectorSubcoreMesh(core_axis_name="c", subcore_axis_name="s")

    @functools.partial(
        pl.kernel,
        mesh=mesh,
        compiler_params=pltpu.CompilerParams(use_tc_tiling_on_sc=False),
        out_type=jax.ShapeDtypeStruct((ROWS, EMBED), jnp.float32),
        scratch_types=[
            pltpu.VMEM((RW // G, G), jnp.int32),
            pltpu.VMEM((MAXLEN, EMBED), jnp.float32),
            pltpu.VMEM((S, EMBED), jnp.float32),
            pltpu.VMEM((S, EMBED), jnp.float32),
            pltpu.SemaphoreType.DMA,
            pltpu.SemaphoreType.DMA,
            pltpu.SemaphoreType.DMA,
            pltpu.SemaphoreType.DMA,
        ],
    )
    def emb_kernel(idx_hbm, tok_hbm, pos_hbm, out_hbm,
                   idx_v, pos_v, rb0, rb1, gs0, gs1, os0, os1):
        wid = lax.axis_index("s") * NC + lax.axis_index("c")
        idx_row0 = wid * (RW // G)
        row0 = wid * RW

        pltpu.sync_copy(idx_hbm.at[pl.ds(idx_row0, RW // G)], idx_v)
        pltpu.sync_copy(pos_hbm, pos_v)

        def fire(g, buf, sem):
            for k in range(NG):
                pltpu.async_copy(
                    tok_hbm.at[idx_v.at[g * NG + k]],
                    buf.at[pl.ds(k * G, G)],
                    sem,
                )

        def drain(buf, sem):
            pltpu.make_async_copy(tok_hbm.at[pl.ds(0, S)], buf, sem).wait()

        def add(buf):
            @plsc.parallel_loop(0, MAXLEN, 1, unroll=2)
            def _(p):
                pv0 = pos_v[p, pl.ds(0, LANES)]
                pv1 = pos_v[p, pl.ds(LANES, LANES)]
                for k in range(S // MAXLEN):
                    r = p + k * MAXLEN
                    buf[r, pl.ds(0, LANES)] = buf[r, pl.ds(0, LANES)] + pv0
                    buf[r, pl.ds(LANES, LANES)] = (
                        buf[r, pl.ds(LANES, LANES)] + pv1
                    )

        def out_fire(g, buf, sem):
            pltpu.async_copy(buf, out_hbm.at[pl.ds(row0 + g * S, S)], sem)

        def out_drain(buf, sem):
            pltpu.make_async_copy(buf, out_hbm.at[pl.ds(row0, S)], sem).wait()

        fire(0, rb0, gs0)
        fire(1, rb1, gs1)

        def body(gg, _):
            g0 = 2 * gg
            drain(rb0, gs0)
            add(rb0)
            out_fire(g0, rb0, os0)
            drain(rb1, gs1)
            add(rb1)
            out_fire(g0 + 1, rb1, os1)
            out_drain(rb0, os0)
            fire(g0 + 2, rb0, gs0)
            out_drain(rb1, os1)
            fire(g0 + 3, rb1, gs1)
            return _

        lax.fori_loop(0, NCH // 2 - 1, body, None)

        drain(rb0, gs0)
        add(rb0)
        out_fire(NCH - 2, rb0, os0)
        drain(rb1, gs1)
        add(rb1)
        out_fire(NCH - 1, rb1, os1)
        out_drain(rb0, os0)
        out_drain(rb1, os1)

    return emb_kernel


_EMB_KERNEL = _make_sc_kernel()


def kernel(inputs, token_table, pos_table):
    idx = inputs.astype(jnp.int32).reshape(ROWS // G, G)
    out = _EMB_KERNEL(idx, token_table, pos_table)
    return out.reshape(BATCH, MAXLEN, EMBED)
